# Initial kernel scaffold; baseline (speedup 1.0000x reference)
import jax
import jax.numpy as jnp
from jax.experimental import pallas as pl


def kernel(x, numbers, weights):
    raise NotImplementedError("write your pallas kernel here")


# speedup vs baseline: 180.9746x; 180.9746x over previous
import functools
import jax
import jax.numpy as jnp
from jax import lax
from jax.experimental import pallas as pl
from jax.experimental.pallas import tpu as pltpu
from jax.experimental.pallas import tpu_sc as plsc

_ROWS, _COLS = 16384, 200
_N = _ROWS * _COLS
_NC, _NS, _L = 2, 16, 16
_NW = _NC * _NS
_PER_W = _N // _NW
_CHUNK = 20480
_NCHUNK = _PER_W // _CHUNK
_TBL = 128

_mesh = plsc.VectorSubcoreMesh(
    core_axis_name="c", subcore_axis_name="s",
    num_cores=_NC, num_subcores=_NS,
)


@functools.partial(
    pl.kernel,
    out_type=jax.ShapeDtypeStruct((_N,), jnp.float32),
    mesh=_mesh,
    compiler_params=pltpu.CompilerParams(needs_layout_passes=False),
    scratch_types=[
        pltpu.VMEM((_TBL,), jnp.float32),
        pltpu.VMEM((_CHUNK,), jnp.float32),
        pltpu.VMEM((_CHUNK,), jnp.int32),
        pltpu.VMEM((_CHUNK,), jnp.float32),
        pltpu.SemaphoreType.DMA,
    ],
)
def _expmult_sc(x_hbm, n_hbm, w_hbm, out_hbm, tbl_v, x_v, n_v, o_v, sem):
    wid = lax.axis_index("s") * _NC + lax.axis_index("c")
    base = wid * _PER_W
    pltpu.sync_copy(w_hbm, tbl_v)
    for c in range(_NCHUNK):
        off = base + c * _CHUNK
        pltpu.sync_copy(x_hbm.at[pl.ds(off, _CHUNK)], x_v)
        pltpu.sync_copy(n_hbm.at[pl.ds(off, _CHUNK)], n_v)

        def body(j, _):
            s = pl.ds(j * _L, _L)
            scale = plsc.load_gather(tbl_v, [n_v[s]])
            o_v[s] = scale * jnp.exp(x_v[s])
            return 0

        lax.fori_loop(0, _CHUNK // _L, body, 0)
        pltpu.sync_copy(o_v, out_hbm.at[pl.ds(off, _CHUNK)])


def kernel(x, numbers, weights):
    out = _expmult_sc(x.reshape(_N), numbers.reshape(_N), weights)
    return out.reshape(_ROWS, _COLS)


# speedup vs baseline: 209.1607x; 1.1557x over previous
import functools
import jax
import jax.numpy as jnp
from jax import lax
from jax.experimental import pallas as pl
from jax.experimental.pallas import tpu as pltpu
from jax.experimental.pallas import tpu_sc as plsc

_ROWS, _COLS = 16384, 200
_N = _ROWS * _COLS
_NC, _NS, _L = 2, 16, 16
_NW = _NC * _NS
_PER_W = _N // _NW
_CHUNK = 20480
_NCHUNK = _PER_W // _CHUNK
_TBL = 128

_mesh = plsc.VectorSubcoreMesh(
    core_axis_name="c", subcore_axis_name="s",
    num_cores=_NC, num_subcores=_NS,
)


@functools.partial(
    pl.kernel,
    out_type=jax.ShapeDtypeStruct((_N,), jnp.float32),
    mesh=_mesh,
    compiler_params=pltpu.CompilerParams(needs_layout_passes=False),
    scratch_types=[
        pltpu.VMEM((_TBL,), jnp.float32),
        pltpu.VMEM((_CHUNK,), jnp.float32),
        pltpu.VMEM((_CHUNK,), jnp.int32),
        pltpu.VMEM((_CHUNK,), jnp.float32),
        pltpu.SemaphoreType.DMA,
    ],
)
def _expmult_sc(x_hbm, n_hbm, w_hbm, out_hbm, tbl_v, x_v, n_v, o_v, sem):
    wid = lax.axis_index("s") * _NC + lax.axis_index("c")
    base = wid * _PER_W
    pltpu.sync_copy(w_hbm, tbl_v)
    for c in range(_NCHUNK):
        off = base + c * _CHUNK
        pltpu.sync_copy(x_hbm.at[pl.ds(off, _CHUNK)], x_v)
        pltpu.sync_copy(n_hbm.at[pl.ds(off, _CHUNK)], n_v)

        @plsc.parallel_loop(0, _CHUNK // _L, unroll=8)
        def body(j):
            s = pl.ds(j * _L, _L)
            scale = plsc.load_gather(tbl_v, [n_v[s]])
            o_v[s] = scale * jnp.exp(x_v[s])
        pltpu.sync_copy(o_v, out_hbm.at[pl.ds(off, _CHUNK)])


def kernel(x, numbers, weights):
    out = _expmult_sc(x.reshape(_N), numbers.reshape(_N), weights)
    return out.reshape(_ROWS, _COLS)


# speedup vs baseline: 227.7952x; 1.0891x over previous
import functools
import jax
import jax.numpy as jnp
from jax import lax
from jax.experimental import pallas as pl
from jax.experimental.pallas import tpu as pltpu
from jax.experimental.pallas import tpu_sc as plsc

_ROWS, _COLS = 16384, 200
_N = _ROWS * _COLS
_NC, _NS, _L = 2, 16, 16
_NW = _NC * _NS
_PER_W = _N // _NW
_CHUNK = 10240
_NCHUNK = _PER_W // _CHUNK
_TBL = 128

_mesh = plsc.VectorSubcoreMesh(
    core_axis_name="c", subcore_axis_name="s",
    num_cores=_NC, num_subcores=_NS,
)


@functools.partial(
    pl.kernel,
    out_type=jax.ShapeDtypeStruct((_N,), jnp.float32),
    mesh=_mesh,
    compiler_params=pltpu.CompilerParams(needs_layout_passes=False),
    scratch_types=[
        pltpu.VMEM((_TBL,), jnp.float32),
        pltpu.VMEM((_CHUNK,), jnp.float32),
        pltpu.VMEM((_CHUNK,), jnp.float32),
        pltpu.VMEM((_CHUNK,), jnp.int32),
        pltpu.VMEM((_CHUNK,), jnp.int32),
        pltpu.VMEM((_CHUNK,), jnp.float32),
        pltpu.VMEM((_CHUNK,), jnp.float32),
        pltpu.SemaphoreType.DMA((2,)),
        pltpu.SemaphoreType.DMA((2,)),
    ],
)
def _expmult_sc(x_hbm, n_hbm, w_hbm, out_hbm, tbl_v, x0_v, x1_v, n0_v, n1_v,
                o0_v, o1_v, in_sem, out_sem):
    wid = lax.axis_index("s") * _NC + lax.axis_index("c")
    base = wid * _PER_W
    pltpu.sync_copy(w_hbm, tbl_v)
    x_bufs = (x0_v, x1_v)
    n_bufs = (n0_v, n1_v)
    o_bufs = (o0_v, o1_v)

    def issue_in(c):
        slot = c % 2
        sl = pl.ds(base + c * _CHUNK, _CHUNK)
        hx = pltpu.async_copy(x_hbm.at[sl], x_bufs[slot], in_sem.at[slot])
        hn = pltpu.async_copy(n_hbm.at[sl], n_bufs[slot], in_sem.at[slot])
        return hx, hn

    in_h = [None, None]
    out_h = [None, None]
    in_h[0] = issue_in(0)

    for c in range(_NCHUNK):
        slot = c % 2
        if c + 1 < _NCHUNK:
            in_h[(c + 1) % 2] = issue_in(c + 1)
        hx, hn = in_h[slot]
        hx.wait()
        hn.wait()
        if out_h[slot] is not None:
            out_h[slot].wait()

        xs = x_bufs[slot]
        ns = n_bufs[slot]
        os_ = o_bufs[slot]

        @plsc.parallel_loop(0, _CHUNK // _L, unroll=8)
        def body(j):
            s = pl.ds(j * _L, _L)
            scale = plsc.load_gather(tbl_v, [ns[s]])
            os_[s] = scale * jnp.exp(xs[s])

        out_h[slot] = pltpu.async_copy(
            o_bufs[slot], out_hbm.at[pl.ds(base + c * _CHUNK, _CHUNK)],
            out_sem.at[slot])

    out_h[(_NCHUNK - 1) % 2].wait()
    out_h[_NCHUNK % 2].wait()


def kernel(x, numbers, weights):
    out = _expmult_sc(x.reshape(_N), numbers.reshape(_N), weights)
    return out.reshape(_ROWS, _COLS)
